# baseline (speedup 1.0000x reference)
import functools

import jax
import jax.numpy as jnp
from jax import lax
from jax.experimental import pallas as pl
from jax.experimental.pallas import tpu as pltpu
from jax.experimental.pallas import tpu_sc as plsc

B = 1024
N = 4096
C_OLD = 23
C_NEW = 11

NUM_WORKERS = 32
SC_K = 2
B_SC = NUM_WORKERS * SC_K * 8
BB_TC = (B - B_SC) // 8
NCH = 256
CH_PER_BB = N // NCH
TOTAL_CH = SC_K * CH_PER_BB
OUT_PER_W = SC_K * 8 * C_NEW * 16
TC_NCH = 512


def _sc_grouped_sums(xt):
    mesh = plsc.VectorSubcoreMesh(core_axis_name="c", subcore_axis_name="s")

    @functools.partial(
        pl.kernel,
        mesh=mesh,
        out_type=jax.ShapeDtypeStruct((NUM_WORKERS, OUT_PER_W), jnp.float32),
        scratch_types=[
            pltpu.VMEM((C_OLD, 8, NCH), jnp.float32),
            pltpu.VMEM((C_OLD, 8, NCH), jnp.float32),
            pltpu.VMEM((OUT_PER_W,), jnp.float32),
            pltpu.SemaphoreType.DMA,
            pltpu.SemaphoreType.DMA,
        ],
        compiler_params=pltpu.CompilerParams(
            needs_layout_passes=False, use_tc_tiling_on_sc=True
        ),
    )
    def k(x_hbm, out_hbm, buf0, buf1, outv, sem0, sem1):
        wid = lax.axis_index("s") * 2 + lax.axis_index("c")
        bufs = (buf0, buf1)
        sems = (sem0, sem1)

        def src(it):
            lb = it // CH_PER_BB
            ch = lax.rem(it, CH_PER_BB)
            b0 = (wid * SC_K + lb) * 8
            return x_hbm.at[:, pl.ds(b0, 8), pl.ds(ch * NCH, NCH)]

        def zero_body(i, _):
            off = pl.multiple_of(i * 16, 16)
            outv[pl.ds(off, 16)] = jnp.zeros((16,), jnp.float32)
            return 0

        lax.fori_loop(0, OUT_PER_W // 16, zero_body, 0)

        for q in range(2):
            pltpu.make_async_copy(src(q), bufs[q], sems[q]).start()

        def compute(buf, it):
            lb = it // CH_PER_BB
            for r in range(8):
                def p_body(p, accs):
                    st = pl.multiple_of(p * 16, 16)
                    cols = [buf[c, r, pl.ds(st, 16)] for c in range(C_OLD)]
                    gsums = [cols[2 * g] + cols[2 * g + 1] for g in range(C_NEW - 1)]
                    gsums.append(cols[20] + cols[21] + cols[22])
                    s = gsums[0]
                    for g in range(1, C_NEW):
                        s = s + gsums[g]
                    w = 1.0 / s
                    return tuple(accs[g] + gsums[g] * w for g in range(C_NEW))

                zeros = tuple(jnp.zeros((16,), jnp.float32) for _ in range(C_NEW))
                accs = plsc.parallel_loop(0, NCH // 16, carry=zeros, unroll=2)(p_body)
                jb = lb * 8 + r
                for g in range(C_NEW):
                    off = pl.multiple_of((jb * C_NEW + g) * 16, 16)
                    plsc.addupdate(outv.at[pl.ds(off, 16)], accs[g])

        def step(s_, _):
            for q in range(2):
                it = 2 * s_ + q
                pltpu.make_async_copy(src(it), bufs[q], sems[q]).wait()
                compute(bufs[q], it)

                @pl.when(it + 2 < TOTAL_CH)
                def _():
                    pltpu.make_async_copy(src(it + 2), bufs[q], sems[q]).start()

            return 0

        lax.fori_loop(0, TOTAL_CH // 2, step, 0)
        pltpu.sync_copy(outv, out_hbm.at[wid])

    return k(xt)


def _tc_grouped_sums(xt):
    bb0 = B_SC // 8

    def body(x_ref, o_ref):
        j = pl.program_id(1)
        x = x_ref[...]
        s = jnp.sum(x, axis=0)
        w = 1.0 / s
        gsums = [x[2 * g] + x[2 * g + 1] for g in range(C_NEW - 1)]
        gsums.append(x[20] + x[21] + x[22])

        @pl.when(j == 0)
        def _():
            o_ref[...] = jnp.zeros_like(o_ref)

        for g in range(C_NEW):
            gw = gsums[g] * w
            part = gw[:, 0:128]
            for k_ in range(1, TC_NCH // 128):
                part = part + gw[:, k_ * 128:(k_ + 1) * 128]
            o_ref[0, g] += part

    return pl.pallas_call(
        body,
        grid=(BB_TC, N // TC_NCH),
        in_specs=[
            pl.BlockSpec((C_OLD, 8, TC_NCH), lambda i, j: (0, bb0 + i, j))
        ],
        out_specs=pl.BlockSpec((1, C_NEW, 8, 128), lambda i, j: (i, 0, 0, 0)),
        out_shape=jax.ShapeDtypeStruct((BB_TC, C_NEW, 8, 128), jnp.float32),
    )(xt)


def _tc_kl_loss(v_sc, v_tc, targets):
    def body(vs_ref, vt_ref, t_ref, o_ref):
        t = t_ref[...]
        ap1 = jnp.sum(vs_ref[...], axis=-1)
        ap2 = jnp.sum(vt_ref[...], axis=-1)
        ap = jnp.concatenate([ap1, ap2], axis=0) * (1.0 / N)
        pw = t * (jnp.log(t) - jnp.log(ap))
        o_ref[0, 0] = jnp.sum(pw) * (1.0 / B)

    out = pl.pallas_call(
        body,
        out_shape=jax.ShapeDtypeStruct((1, 1), jnp.float32),
        out_specs=pl.BlockSpec(memory_space=pltpu.SMEM),
    )(v_sc, v_tc, targets)
    return out[0, 0]


@jax.jit
def kernel(inputs, targets):
    xt = jnp.moveaxis(inputs, -1, 0)
    v_sc = _sc_grouped_sums(xt).reshape(B_SC, C_NEW, 16)
    v_tc = jnp.moveaxis(_tc_grouped_sums(xt), 1, 2).reshape(B - B_SC, C_NEW, 128)
    return _tc_kl_loss(v_sc, v_tc, targets)


# speedup vs baseline: 2.1340x; 2.1340x over previous
import functools

import jax
import jax.numpy as jnp
from jax import lax
from jax.experimental import pallas as pl
from jax.experimental.pallas import tpu as pltpu
from jax.experimental.pallas import tpu_sc as plsc

B = 1024
N = 4096
C_OLD = 23
C_NEW = 11

NUM_WORKERS = 32
SC_K = 2
B_SC = NUM_WORKERS * SC_K * 8
BB_TC = (B - B_SC) // 8
NCH = 256
CH_PER_BB = N // NCH
TOTAL_CH = SC_K * CH_PER_BB
OUT_PER_W = SC_K * 8 * C_NEW * 16
TC_NCH = 512


def _sc_grouped_sums(xt):
    mesh = plsc.VectorSubcoreMesh(core_axis_name="c", subcore_axis_name="s")

    @functools.partial(
        pl.kernel,
        mesh=mesh,
        out_type=jax.ShapeDtypeStruct((NUM_WORKERS, OUT_PER_W), jnp.float32),
        scratch_types=[
            pltpu.VMEM((C_OLD, 8, NCH), jnp.float32),
            pltpu.VMEM((C_OLD, 8, NCH), jnp.float32),
            pltpu.VMEM((OUT_PER_W,), jnp.float32),
            pltpu.SemaphoreType.DMA,
            pltpu.SemaphoreType.DMA,
        ],
        compiler_params=pltpu.CompilerParams(
            needs_layout_passes=False, use_tc_tiling_on_sc=True
        ),
    )
    def k(x_hbm, out_hbm, buf0, buf1, outv, sem0, sem1):
        wid = lax.axis_index("s") * 2 + lax.axis_index("c")
        bufs = (buf0, buf1)
        sems = (sem0, sem1)

        def src(it):
            lb = it // CH_PER_BB
            ch = lax.rem(it, CH_PER_BB)
            b0 = (wid * SC_K + lb) * 8
            return x_hbm.at[:, pl.ds(b0, 8), pl.ds(ch * NCH, NCH)]

        def zero_body(i, _):
            off = pl.multiple_of(i * 16, 16)
            outv[pl.ds(off, 16)] = jnp.zeros((16,), jnp.float32)
            return 0

        lax.fori_loop(0, OUT_PER_W // 16, zero_body, 0)

        for q in range(2):
            pltpu.make_async_copy(src(q), bufs[q], sems[q]).start()

        def compute(buf, it):
            lb = it // CH_PER_BB
            for r in range(8):
                def p_body(p, accs):
                    st = pl.multiple_of(p * 16, 16)
                    cols = [buf[c, r, pl.ds(st, 16)] for c in range(C_OLD)]
                    gsums = [cols[2 * g] + cols[2 * g + 1] for g in range(C_NEW - 1)]
                    gsums.append(cols[20] + cols[21] + cols[22])
                    s = gsums[0]
                    for g in range(1, C_NEW):
                        s = s + gsums[g]
                    w = 1.0 / s
                    return tuple(accs[g] + gsums[g] * w for g in range(C_NEW))

                zeros = tuple(jnp.zeros((16,), jnp.float32) for _ in range(C_NEW))
                accs = plsc.parallel_loop(0, NCH // 16, carry=zeros, unroll=2)(p_body)
                jb = lb * 8 + r
                for g in range(C_NEW):
                    off = pl.multiple_of((jb * C_NEW + g) * 16, 16)
                    plsc.addupdate(outv.at[pl.ds(off, 16)], accs[g])

        def step(s_, _):
            for q in range(2):
                it = 2 * s_ + q
                pltpu.make_async_copy(src(it), bufs[q], sems[q]).wait()
                compute(bufs[q], it)

                @pl.when(it + 2 < TOTAL_CH)
                def _():
                    pltpu.make_async_copy(src(it + 2), bufs[q], sems[q]).start()

            return 0

        lax.fori_loop(0, TOTAL_CH // 2, step, 0)
        pltpu.sync_copy(outv, out_hbm.at[wid])

    return k(xt)


def _tc_grouped_sums(xt):
    bb0 = B_SC // 8

    def body(x_ref, o_ref):
        x = x_ref[...]
        s = jnp.sum(x, axis=0)
        w = 1.0 / s
        gsums = [x[2 * g] + x[2 * g + 1] for g in range(C_NEW - 1)]
        gsums.append(x[20] + x[21] + x[22])
        for g in range(C_NEW):
            gw = (gsums[g] * w).reshape(8, N // 128, 128)
            o_ref[0, g] = jnp.sum(gw, axis=1)

    return pl.pallas_call(
        body,
        grid=(BB_TC,),
        in_specs=[pl.BlockSpec((C_OLD, 8, N), lambda i: (0, bb0 + i, 0))],
        out_specs=pl.BlockSpec((1, C_NEW, 8, 128), lambda i: (i, 0, 0, 0)),
        out_shape=jax.ShapeDtypeStruct((BB_TC, C_NEW, 8, 128), jnp.float32),
    )(xt)


def _tc_kl_loss(v_sc, v_tc, targets):
    def body(vs_ref, vt_ref, t_ref, o_ref):
        t = t_ref[...]
        ap1 = jnp.sum(vs_ref[...], axis=-1)
        ap2 = jnp.sum(vt_ref[...], axis=-1)
        ap = jnp.concatenate([ap1, ap2], axis=0) * (1.0 / N)
        pw = t * (jnp.log(t) - jnp.log(ap))
        o_ref[0, 0] = jnp.sum(pw) * (1.0 / B)

    out = pl.pallas_call(
        body,
        out_shape=jax.ShapeDtypeStruct((1, 1), jnp.float32),
        out_specs=pl.BlockSpec(memory_space=pltpu.SMEM),
    )(v_sc, v_tc, targets)
    return out[0, 0]


@jax.jit
def kernel(inputs, targets):
    xt = jnp.moveaxis(inputs, -1, 0)
    v_sc = _sc_grouped_sums(xt).reshape(B_SC, C_NEW, 16)
    v_tc = jnp.moveaxis(_tc_grouped_sums(xt), 1, 2).reshape(B - B_SC, C_NEW, 128)
    return _tc_kl_loss(v_sc, v_tc, targets)
